# baseline (speedup 1.0000x reference)
import functools

import jax
import jax.numpy as jnp
from jax import lax
from jax.experimental import pallas as pl
from jax.experimental.pallas import tpu as pltpu
from jax.experimental.pallas import tpu_sc as plsc

NUM_TOKENS = 4096 * 200
DIM = 128

_CHUNK = 128
_CYCLE = 5
_NBUF = 4
_NSLOT = 2


def _build():
    info = plsc.get_sparse_core_info()
    nw = info.num_cores * info.num_subcores
    rows_per_w = NUM_TOKENS // nw
    n_chunks = rows_per_w // _CHUNK
    n_groups = n_chunks // _CYCLE
    idx_rows_per_w = n_chunks

    mesh = plsc.VectorSubcoreMesh(core_axis_name="c", subcore_axis_name="s")

    @functools.partial(
        pl.kernel,
        mesh=mesh,
        out_type=jax.ShapeDtypeStruct((NUM_TOKENS, DIM), jnp.float32),
        scratch_types=[
            pltpu.VMEM((idx_rows_per_w, _CHUNK), jnp.int32),
            pltpu.VMEM((_NBUF, _CHUNK, DIM), jnp.float32),
            pltpu.VMEM_SHARED((info.num_subcores, _NSLOT, _CHUNK, DIM),
                              jnp.float32),
        ] + [pltpu.SemaphoreType.DMA] * (_NBUF + 2 * _NSLOT + 1),
    )
    def emb(idx_hbm, table_hbm, out_hbm, idx_v, rows_v, sp, *sems):
        gsems = sems[:_NBUF]
        csems = sems[_NBUF:_NBUF + _NSLOT]
        hsems = sems[_NBUF + _NSLOT:_NBUF + 2 * _NSLOT]
        psem = sems[-1]

        wid = lax.axis_index("s") * info.num_cores + lax.axis_index("c")
        sid = lax.axis_index("s")
        base = wid * rows_per_w

        pltpu.sync_copy(idx_hbm.at[pl.ds(wid * idx_rows_per_w, idx_rows_per_w)],
                        idx_v)

        def out_at(j):
            return out_hbm.at[pl.ds(base + j * _CHUNK, _CHUNK)]

        def gather(j, b):
            return pltpu.make_async_copy(
                table_hbm.at[idx_v.at[j]], rows_v.at[b], gsems[b])

        def xbar(b, s):
            return pltpu.make_async_copy(rows_v.at[b], sp.at[sid, s], csems[s])

        def drain_sp(j, s):
            return pltpu.make_async_copy(sp.at[sid, s], out_at(j), hsems[s])

        def put_port(j, b):
            return pltpu.make_async_copy(rows_v.at[b], out_at(j), psem)

        def do_group(j0, first):
            if not first:
                drain_sp(j0 - _CYCLE + 2, 0).wait()
                put_port(j0 - 1, 0).wait()
            gather(j0 + 0, 0).start()
            if not first:
                drain_sp(j0 - _CYCLE + 3, 1).wait()
            gather(j0 + 1, 1).start()
            gather(j0 + 2, 2).start()
            gather(j0 + 3, 3).start()

            gather(j0 + 0, 0).wait()
            xbar(0, 0).start()
            xbar(0, 0).wait()
            drain_sp(j0 + 0, 0).start()
            gather(j0 + 4, 0).start()

            gather(j0 + 1, 1).wait()
            xbar(1, 1).start()
            xbar(1, 1).wait()
            drain_sp(j0 + 1, 1).start()

            gather(j0 + 2, 2).wait()
            drain_sp(j0 + 0, 0).wait()
            xbar(2, 0).start()
            xbar(2, 0).wait()
            drain_sp(j0 + 2, 0).start()

            gather(j0 + 3, 3).wait()
            drain_sp(j0 + 1, 1).wait()
            xbar(3, 1).start()
            xbar(3, 1).wait()
            drain_sp(j0 + 3, 1).start()

            gather(j0 + 4, 0).wait()
            put_port(j0 + 4, 0).start()

        do_group(0, first=True)

        def group(g, _):
            do_group(g * _CYCLE, first=False)
            return _

        lax.fori_loop(1, n_groups, group, None)

        j0 = (n_groups - 1) * _CYCLE
        drain_sp(j0 + 2, 0).wait()
        drain_sp(j0 + 3, 1).wait()
        put_port(j0 + 4, 0).wait()

    return emb


_EMB = _build()


@jax.jit
def kernel(token_ids, weight):
    idx2d = token_ids.reshape(NUM_TOKENS // _CHUNK, _CHUNK).astype(jnp.int32)
    out = _EMB(idx2d, weight)
    return out.reshape(*token_ids.shape, DIM)
